# Initial kernel scaffold; baseline (speedup 1.0000x reference)
import jax
import jax.numpy as jnp
from jax.experimental import pallas as pl


def kernel(x, W):
    raise NotImplementedError("write your pallas kernel here")


# speedup vs baseline: 1.9171x; 1.9171x over previous
import math

import jax
import jax.numpy as jnp
from jax.experimental import pallas as pl


BLOCK_L = 512


def _pe_add_kernel(x_ref, w_ref, o_ref):
    w = w_ref[...]
    d_model = w.shape[-1]
    max_norm = math.sqrt(float(d_model))
    norm_sq = jnp.sum(w * w, axis=-1, keepdims=True)
    norm = jnp.sqrt(norm_sq)
    scale = jnp.minimum(1.0, max_norm / jnp.maximum(norm, 1e-12))
    pe = w * scale
    o_ref[...] = x_ref[...] + pe[None, :, :]


def kernel(x, W):
    batch, seq_len, d_model = x.shape
    block_l = min(BLOCK_L, seq_len)
    grid = (seq_len // block_l,)
    return pl.pallas_call(
        _pe_add_kernel,
        grid=grid,
        in_specs=[
            pl.BlockSpec((batch, block_l, d_model), lambda i: (0, i, 0)),
            pl.BlockSpec((block_l, d_model), lambda i: (i, 0)),
        ],
        out_specs=pl.BlockSpec((batch, block_l, d_model), lambda i: (0, i, 0)),
        out_shape=jax.ShapeDtypeStruct(x.shape, x.dtype),
    )(x, W)
